# baseline (speedup 1.0000x reference)
import functools

import jax
import jax.numpy as jnp
from jax import lax
from jax.experimental import pallas as pl
from jax.experimental.pallas import tpu as pltpu
from jax.experimental.pallas import tpu_sc as plsc

_NC = 2
_NS = 16
_NW = _NC * _NS
_BB = 128
_NSLOT = 3


def _gather_call(S, B, D, units_per_w):
    mesh = plsc.VectorSubcoreMesh(core_axis_name="c", subcore_axis_name="s")
    DT = D // 8
    CB = B // _BB

    @functools.partial(
        pl.kernel,
        out_type=jax.ShapeDtypeStruct((S, DT, CB, 8, _BB), jnp.float32),
        mesh=mesh,
        compiler_params=pltpu.CompilerParams(
            use_tc_tiling_on_sc=False, needs_layout_passes=False
        ),
        scratch_types=[
            [pltpu.VMEM((_BB,), jnp.int32)] * _NSLOT,
            [pltpu.VMEM((_BB, D), jnp.float32)] * _NSLOT,
            [pltpu.VMEM((DT, 8, _BB), jnp.float32)] * _NSLOT,
            [pltpu.SemaphoreType.DMA] * _NSLOT,
            [pltpu.SemaphoreType.DMA] * _NSLOT,
            [pltpu.SemaphoreType.DMA] * _NSLOT,
        ],
    )
    def k(idx_hbm, table_hbm, out_hbm, idxs, gbufs, obufs, isems, gsems, wsems):
        wid = lax.axis_index("s") * _NC + lax.axis_index("c")
        u0 = wid * units_per_w

        def unit_sc(u):
            return u // CB, u % CB

        def idx_start(u, p):
            s, c = unit_sc(u)
            pltpu.async_copy(
                idx_hbm.at[s, pl.ds(c * _BB, _BB)], idxs[p], isems[p]
            )

        def gather_start(u, p):
            pltpu.make_async_copy(
                idx_hbm.at[0, pl.ds(0, _BB)], idxs[p], isems[p]
            ).wait()
            pltpu.async_copy(table_hbm.at[idxs[p]], gbufs[p], gsems[p])

        def write_start(u, p):
            s, c = unit_sc(u)
            pltpu.async_copy(
                obufs[p], out_hbm.at[s, :, c, :, :], wsems[p]
            )

        def write_wait(p):
            pltpu.make_async_copy(
                obufs[p], out_hbm.at[0, :, 0, :, :], wsems[p]
            ).wait()

        def transform(p):
            pltpu.make_async_copy(
                table_hbm.at[pl.ds(0, _BB), :], gbufs[p], gsems[p]
            ).wait()
            for q in range(_BB // 16):
                rows = lax.iota(jnp.int32, 16) + (16 * q)
                for d in range(D):
                    cols = jnp.full((16,), d, jnp.int32)
                    v = plsc.load_gather(gbufs[p], [rows, cols])
                    obufs[p][d // 8, d % 8, pl.ds(16 * q, 16)] = v

        idx_start(u0, 0)
        idx_start(u0 + 1, 1)
        gather_start(u0, 0)

        n_iter = units_per_w // _NSLOT

        def step(h, carry):
            g = h * _NSLOT
            for b in range(_NSLOT):
                u = u0 + g + b
                p = b
                pn = (b + 1) % _NSLOT
                pn2 = (b + 2) % _NSLOT
                idx_start(u + 2, pn2)
                gather_start(u + 1, pn)

                @pl.when(g + b >= _NSLOT)
                def _():
                    write_wait(p)

                transform(p)
                write_start(u, p)
            return carry

        lax.fori_loop(0, n_iter, step, 0)

        for r in range(_NSLOT * n_iter, units_per_w):
            u = u0 + r
            p = r % _NSLOT
            if r + 1 < units_per_w:
                gather_start(u + 1, (r + 1) % _NSLOT)
            write_wait(p)
            transform(p)
            write_start(u, p)
        for p in range(_NSLOT):
            write_wait(p)

    return k


def kernel(token_ids, weight):
    B, S = token_ids.shape
    D = weight.shape[1]
    units = S * (B // _BB)
    units_per_w = units // _NW
    idx_t = token_ids.T.astype(jnp.int32)
    o5 = _gather_call(S, B, D, units_per_w)(idx_t, weight)
    return o5.transpose(2, 4, 0, 1, 3).reshape(B, S, D)


# speedup vs baseline: 1.2394x; 1.2394x over previous
import functools

import jax
import jax.numpy as jnp
from jax import lax
from jax.experimental import pallas as pl
from jax.experimental.pallas import tpu as pltpu
from jax.experimental.pallas import tpu_sc as plsc

_NC = 2
_NS = 16
_NW = _NC * _NS
_BB = 128
_NSLOT = 3


def _gather_call(S, B, D, units_per_w):
    mesh = plsc.VectorSubcoreMesh(core_axis_name="c", subcore_axis_name="s")
    DT = D // 8
    CB = B // _BB

    @functools.partial(
        pl.kernel,
        out_type=jax.ShapeDtypeStruct((S, DT, CB, 8, _BB), jnp.float32),
        mesh=mesh,
        compiler_params=pltpu.CompilerParams(
            use_tc_tiling_on_sc=False, needs_layout_passes=False
        ),
        scratch_types=[
            [pltpu.VMEM((_BB,), jnp.int32)] * _NSLOT,
            [pltpu.VMEM((_BB, D), jnp.float32)] * _NSLOT,
            [pltpu.VMEM((DT, 8, _BB), jnp.float32)] * _NSLOT,
            [pltpu.SemaphoreType.DMA] * _NSLOT,
            [pltpu.SemaphoreType.DMA] * _NSLOT,
            [pltpu.SemaphoreType.DMA] * _NSLOT,
        ],
    )
    def k(idx_hbm, table_hbm, out_hbm, idxs, gbufs, obufs, isems, gsems, wsems):
        wid = lax.axis_index("s") * _NC + lax.axis_index("c")
        u0 = wid * units_per_w

        def unit_sc(u):
            return u // CB, u % CB

        def idx_start(u, p):
            s, c = unit_sc(u)
            pltpu.async_copy(
                idx_hbm.at[s, pl.ds(c * _BB, _BB)], idxs[p], isems[p]
            )

        def gather_start(u, p):
            pltpu.make_async_copy(
                idx_hbm.at[0, pl.ds(0, _BB)], idxs[p], isems[p]
            ).wait()
            pltpu.async_copy(table_hbm.at[idxs[p]], gbufs[p], gsems[p])

        def write_start(u, p):
            s, c = unit_sc(u)
            pltpu.async_copy(
                obufs[p], out_hbm.at[s, :, c, :, :], wsems[p]
            )

        def write_wait(p):
            pltpu.make_async_copy(
                obufs[p], out_hbm.at[0, :, 0, :, :], wsems[p]
            ).wait()

        def transform(p):
            pltpu.make_async_copy(
                table_hbm.at[pl.ds(0, _BB), :], gbufs[p], gsems[p]
            ).wait()
            rows = [lax.iota(jnp.int32, 16) + (16 * q) for q in range(_BB // 16)]
            pairs = [(q, d) for d in range(D) for q in range(_BB // 16)]
            for i in range(0, len(pairs), 16):
                chunk = pairs[i : i + 16]
                vals = [
                    plsc.load_gather(
                        gbufs[p], [rows[q], jnp.full((16,), d, jnp.int32)]
                    )
                    for (q, d) in chunk
                ]
                for (q, d), v in zip(chunk, vals):
                    obufs[p][d // 8, d % 8, pl.ds(16 * q, 16)] = v

        for p in range(_NSLOT):
            idx_start(u0 + p, p)
        gather_start(u0, 0)
        gather_start(u0 + 1, 1)

        n_iter = (units_per_w - 2) // _NSLOT

        def step(h, carry):
            g = h * _NSLOT
            for b in range(_NSLOT):
                u = u0 + g + b
                p = b
                pn2 = (b + 2) % _NSLOT
                gather_start(u + 2, pn2)

                @pl.when(g + b >= _NSLOT)
                def _():
                    write_wait(p)

                transform(p)
                write_start(u, p)

                @pl.when(g + b + _NSLOT < units_per_w)
                def _():
                    idx_start(u + _NSLOT, p)

            return carry

        lax.fori_loop(0, n_iter, step, 0)

        for r in range(_NSLOT * n_iter, units_per_w):
            u = u0 + r
            p = r % _NSLOT
            write_wait(p)
            transform(p)
            write_start(u, p)
        for p in range(_NSLOT):
            write_wait(p)

    return k


def kernel(token_ids, weight):
    B, S = token_ids.shape
    D = weight.shape[1]
    units = S * (B // _BB)
    units_per_w = units // _NW
    idx_t = token_ids.T.astype(jnp.int32)
    o5 = _gather_call(S, B, D, units_per_w)(idx_t, weight)
    return o5.transpose(2, 4, 0, 1, 3).reshape(B, S, D)


# speedup vs baseline: 1.6311x; 1.3160x over previous
import functools

import jax
import jax.numpy as jnp
from jax import lax
from jax.experimental import pallas as pl
from jax.experimental.pallas import tpu as pltpu
from jax.experimental.pallas import tpu_sc as plsc

_NC = 2
_NS = 16
_NW = _NC * _NS
_BB = 128
_NSLOT = 3


def _gather_call(S, B, D, units_per_w):
    mesh = plsc.VectorSubcoreMesh(core_axis_name="c", subcore_axis_name="s")
    DT = D // 8
    CB = B // _BB

    @functools.partial(
        pl.kernel,
        out_type=jax.ShapeDtypeStruct((S, DT, CB, 8, _BB), jnp.float32),
        mesh=mesh,
        compiler_params=pltpu.CompilerParams(
            use_tc_tiling_on_sc=False, needs_layout_passes=False
        ),
        scratch_types=[
            [pltpu.VMEM((_BB,), jnp.int32)] * _NSLOT,
            [pltpu.VMEM((_BB, D), jnp.float32)] * _NSLOT,
            [pltpu.VMEM((DT, 8, _BB + 1), jnp.float32)] * _NSLOT,
            [pltpu.SemaphoreType.DMA] * _NSLOT,
            [pltpu.SemaphoreType.DMA] * _NSLOT,
            [pltpu.SemaphoreType.DMA] * _NSLOT,
        ],
    )
    def k(idx_hbm, table_hbm, out_hbm, idxs, gbufs, obufs, isems, gsems, wsems):
        wid = lax.axis_index("s") * _NC + lax.axis_index("c")
        u0 = wid * units_per_w

        def unit_sc(u):
            return u // CB, u % CB

        def idx_start(u, p):
            s, c = unit_sc(u)
            pltpu.async_copy(
                idx_hbm.at[s, pl.ds(c * _BB, _BB)], idxs[p], isems[p]
            )

        def gather_start(u, p):
            pltpu.make_async_copy(
                idx_hbm.at[0, pl.ds(0, _BB)], idxs[p], isems[p]
            ).wait()
            pltpu.async_copy(table_hbm.at[idxs[p]], gbufs[p], gsems[p])

        def write_start(u, p):
            s, c = unit_sc(u)
            pltpu.async_copy(
                obufs[p].at[:, :, pl.ds(0, _BB)],
                out_hbm.at[s, :, c, :, :],
                wsems[p],
            )

        def write_wait(p):
            pltpu.make_async_copy(
                obufs[p].at[:, :, pl.ds(0, _BB)],
                out_hbm.at[0, :, 0, :, :],
                wsems[p],
            ).wait()

        def transform(p):
            pltpu.make_async_copy(
                table_hbm.at[pl.ds(0, _BB), :], gbufs[p], gsems[p]
            ).wait()
            lanes = lax.iota(jnp.int32, 16)
            i0 = [(lanes + 16 * kk) >> 3 for kk in range(D // 16)]
            i1 = [(lanes + 16 * kk) & 7 for kk in range(D // 16)]
            pairs = [(kk, bb) for kk in range(D // 16) for bb in range(_BB)]
            for i in range(0, len(pairs), 16):
                chunk = pairs[i : i + 16]
                vals = [
                    gbufs[p][bb, pl.ds(16 * kk, 16)] for (kk, bb) in chunk
                ]
                for (kk, bb), v in zip(chunk, vals):
                    plsc.store_scatter(
                        obufs[p],
                        [i0[kk], i1[kk], jnp.full((16,), bb, jnp.int32)],
                        v,
                    )

        for p in range(_NSLOT):
            idx_start(u0 + p, p)
        gather_start(u0, 0)
        gather_start(u0 + 1, 1)

        n_iter = (units_per_w - 2) // _NSLOT

        def step(h, carry):
            g = h * _NSLOT
            for b in range(_NSLOT):
                u = u0 + g + b
                p = b
                pn2 = (b + 2) % _NSLOT
                gather_start(u + 2, pn2)

                @pl.when(g + b >= _NSLOT)
                def _():
                    write_wait(p)

                transform(p)
                write_start(u, p)

                @pl.when(g + b + _NSLOT < units_per_w)
                def _():
                    idx_start(u + _NSLOT, p)

            return carry

        lax.fori_loop(0, n_iter, step, 0)

        for r in range(_NSLOT * n_iter, units_per_w):
            u = u0 + r
            p = r % _NSLOT
            write_wait(p)
            transform(p)
            write_start(u, p)
        for p in range(_NSLOT):
            write_wait(p)

    return k


def kernel(token_ids, weight):
    B, S = token_ids.shape
    D = weight.shape[1]
    units = S * (B // _BB)
    units_per_w = units // _NW
    idx_t = token_ids.T.astype(jnp.int32)
    o5 = _gather_call(S, B, D, units_per_w)(idx_t, weight)
    return o5.transpose(2, 4, 0, 1, 3).reshape(B, S, D)
